# baseline (speedup 1.0000x reference)
import jax
import jax.numpy as jnp
from jax.experimental import pallas as pl
from jax.experimental.pallas import tpu as pltpu

S_TILE = 1024


def _enc_kernel(x_ref, yang_ref, yin_ref, pe_ref, o_ref, sig_ref):
    i = pl.program_id(0)
    j = pl.program_id(1)

    @pl.when(j == 0)
    def _():
        base = i * S_TILE
        pos = base + jax.lax.broadcasted_iota(jnp.int32, (S_TILE, 12), 0)
        col = jax.lax.broadcasted_iota(jnp.int32, (S_TILE, 12), 1)
        yang_oh = (pos % 12 == col).astype(jnp.float32)
        yin_oh = ((pos + 6) % 12 == col).astype(jnp.float32)
        yang = jnp.dot(yang_oh, yang_ref[...], preferred_element_type=jnp.float32)
        yin = jnp.dot(yin_oh, yin_ref[...], preferred_element_type=jnp.float32)
        sig_ref[...] = jnp.concatenate([yang, yin], axis=-1) + pe_ref[...]

    o_ref[...] = x_ref[...] + sig_ref[...][None]


def kernel(x, yang_wheel, yin_wheel, grand_cycle_pe):
    b, s, d = x.shape
    half = yang_wheel.shape[1]
    assert s % S_TILE == 0 and d == 2 * half
    n_tiles = s // S_TILE

    return pl.pallas_call(
        _enc_kernel,
        grid=(n_tiles, b),
        in_specs=[
            pl.BlockSpec((1, S_TILE, d), lambda i, j: (j, i, 0)),
            pl.BlockSpec(yang_wheel.shape, lambda i, j: (0, 0)),
            pl.BlockSpec(yin_wheel.shape, lambda i, j: (0, 0)),
            pl.BlockSpec((S_TILE, d), lambda i, j: (i, 0)),
        ],
        out_specs=pl.BlockSpec((1, S_TILE, d), lambda i, j: (j, i, 0)),
        out_shape=jax.ShapeDtypeStruct((b, s, d), x.dtype),
        scratch_shapes=[pltpu.VMEM((S_TILE, d), jnp.float32)],
        compiler_params=pltpu.CompilerParams(
            dimension_semantics=("arbitrary", "arbitrary"),
        ),
    )(x, yang_wheel, yin_wheel, grand_cycle_pe)
